# baseline (speedup 1.0000x reference)
import math

import jax
import jax.numpy as jnp
from jax import lax
from jax.experimental import pallas as pl
from jax.experimental.pallas import tpu as pltpu
from jax.experimental.pallas import tpu_sc as plsc

N = 10000
D = 128
E = 320000
EPS_BN = 1e-12
BN_SCALE = 1.0 / math.sqrt(1.0 + EPS_BN)

NC, NS, L = 2, 16, 16
NW = NC * NS
K = 112
CPT_H = 120
CPT_L = 60
CPT = CPT_H
HEAVY = 0
E_PAD = NS * (CPT_H + CPT_L) * K
RPT = 640
NPAD = NS * RPT
NBUF = 3
SR = 6
BR = 400


def _sc_w_body(asrc_hbm, adst_hbm, src_hbm, dst_hbm,
               w_hbm, denh_hbm,
               asrc_v, adst_v, src_v, dst_v, w_v, den_l):
    c = lax.axis_index("c")
    s = lax.axis_index("s")
    wid = c * NS + s

    pltpu.sync_copy(asrc_hbm, asrc_v)
    pltpu.sync_copy(adst_hbm, adst_v)
    pltpu.sync_copy(src_hbm.at[wid], src_v)
    pltpu.sync_copy(dst_hbm.at[wid], dst_v)

    zero16 = jnp.zeros((L,), jnp.float32)

    def zbody(i, carry):
        den_l[pl.ds(i * L, L)] = zero16
        return carry
    lax.fori_loop(0, NPAD // L, zbody, 0)

    cpt = jnp.where(c == HEAVY, CPT_H, CPT_L)

    def wbody(j, carry):
        @pl.when(j < cpt)
        def _():
            for v in range(K // L):
                sl = pl.ds(v * L, L)
                si = src_v[j, sl]
                di = dst_v[j, sl]
                a = (plsc.load_gather(asrc_v, [si])
                     + plsc.load_gather(adst_v, [di]))
                w = jnp.exp(jnp.maximum(a, 0.2 * a))
                w_v[j, sl] = w
                plsc.addupdate_scatter(den_l, [di], w)
        return carry
    lax.fori_loop(0, CPT, wbody, 0)

    pltpu.sync_copy(w_v, w_hbm.at[wid])
    pltpu.sync_copy(den_l, denh_hbm.at[wid])


_sc_w = pl.kernel(
    _sc_w_body,
    out_type=(jax.ShapeDtypeStruct((NW, CPT, K), jnp.float32),
              jax.ShapeDtypeStruct((NW, NPAD), jnp.float32)),
    mesh=plsc.VectorSubcoreMesh(core_axis_name="c", subcore_axis_name="s",
                                num_cores=NC, num_subcores=NS),
    compiler_params=pltpu.CompilerParams(needs_layout_passes=False),
    scratch_types=[
        pltpu.VMEM((NPAD,), jnp.float32),
        pltpu.VMEM((NPAD,), jnp.float32),
        pltpu.VMEM((CPT, K), jnp.int32),
        pltpu.VMEM((CPT, K), jnp.int32),
        pltpu.VMEM((CPT, K), jnp.float32),
        pltpu.VMEM((NPAD,), jnp.float32),
    ],
)


def _sc_scatter_body(h_hbm, src_hbm, dst_hbm, w_hbm, z_hbm,
                     outp_hbm,
                     src_r, dst_r, w_r, hbuf, out_acc, *sems):
    gsem = sems[0:NBUF]
    ssem = sems[NBUF:2 * NBUF]
    stsem = sems[2 * NBUF:2 * NBUF + SR]
    c = lax.axis_index("c")
    s = lax.axis_index("s")
    wid = c * NS + s
    r0 = s * RPT
    cpt = jnp.where(c == HEAVY, CPT_H, CPT_L)

    pltpu.sync_copy(z_hbm.at[pl.ds(r0, RPT)], out_acc.at[pl.ds(r0, RPT)])
    plsc.subcore_barrier()

    def stage(j, sr):
        pltpu.async_copy(src_hbm.at[wid, j], src_r.at[sr], stsem[sr])
        pltpu.async_copy(dst_hbm.at[wid, j], dst_r.at[sr], stsem[sr])
        pltpu.async_copy(w_hbm.at[wid, j], w_r.at[sr], stsem[sr])

    def wait_stage(j, sr):
        pltpu.make_async_copy(src_hbm.at[wid, j], src_r.at[sr], stsem[sr]).wait()
        pltpu.make_async_copy(dst_hbm.at[wid, j], dst_r.at[sr], stsem[sr]).wait()
        pltpu.make_async_copy(w_hbm.at[wid, j], w_r.at[sr], stsem[sr]).wait()

    def chunk(j, b, sr):
        bn = (b + 2) % NBUF

        @pl.when(j < cpt)
        def _():
            pltpu.make_async_copy(h_hbm.at[src_r.at[sr]], hbuf.at[b],
                                  gsem[b]).wait()

            def sbody(g, carry):
                wv = w_r[sr, pl.ds(g * L, L)]
                for t in range(L):
                    wk = wv[t]
                    row = g * L + t
                    for u in range(D // L):
                        sl = pl.ds(u * L, L)
                        hbuf[b, row, sl] = hbuf[b, row, sl] * wk
                return carry
            lax.fori_loop(0, K // L, sbody, 0)

        @pl.when((j >= 1) & (j < cpt))
        def _():
            pltpu.make_async_copy(hbuf.at[bn],
                                  out_acc.at[dst_r.at[(sr + SR - 1) % SR]],
                                  ssem[bn]).wait()

        @pl.when(j + 3 < cpt)
        def _():
            stage(j + 3, (sr + 3) % SR)

        @pl.when(j + 2 < cpt)
        def _():
            wait_stage(j + 2, (sr + 2) % SR)
            pltpu.async_copy(h_hbm.at[src_r.at[(sr + 2) % SR]], hbuf.at[bn],
                             gsem[bn])

        @pl.when(j < cpt)
        def _():
            pltpu.async_copy(hbuf.at[b], out_acc.at[dst_r.at[sr]], ssem[b],
                             add=True)

    stage(0, 0)
    stage(1, 1)
    stage(2, 2)
    wait_stage(0, 0)
    pltpu.async_copy(h_hbm.at[src_r.at[0]], hbuf.at[0], gsem[0])
    wait_stage(1, 1)
    pltpu.async_copy(h_hbm.at[src_r.at[1]], hbuf.at[1], gsem[1])

    def cbody(t, carry):
        j = SR * t
        for k in range(SR):
            chunk(j + k, k % NBUF, k)
        return carry
    lax.fori_loop(0, CPT // SR, cbody, 0)

    assert (CPT_H - 1) % NBUF == (CPT_L - 1) % NBUF
    assert (CPT_H - 1) % SR == (CPT_L - 1) % SR
    pltpu.make_async_copy(hbuf.at[(CPT_H - 1) % NBUF],
                          out_acc.at[dst_r.at[(CPT_H - 1) % SR]],
                          ssem[(CPT_H - 1) % NBUF]).wait()
    plsc.subcore_barrier()

    pltpu.sync_copy(out_acc.at[pl.ds(r0, RPT)], outp_hbm.at[c, pl.ds(r0, RPT)])


_sc_scatter = pl.kernel(
    _sc_scatter_body,
    out_type=jax.ShapeDtypeStruct((NC, NPAD, D), jnp.float32),
    mesh=plsc.VectorSubcoreMesh(core_axis_name="c", subcore_axis_name="s",
                                num_cores=NC, num_subcores=NS),
    compiler_params=pltpu.CompilerParams(needs_layout_passes=False),
    scratch_types=[
        pltpu.VMEM((SR, K), jnp.int32),
        pltpu.VMEM((SR, K), jnp.int32),
        pltpu.VMEM((SR, K), jnp.float32),
        pltpu.VMEM((NBUF, K, D), jnp.float32),
        pltpu.VMEM_SHARED((NPAD, D), jnp.float32),
    ] + [pltpu.SemaphoreType.DMA] * (2 * NBUF + SR),
)


def _prep_body(x_ref, w_ref, a8_ref, h_ref, a2_ref):
    h = jnp.dot(x_ref[...], w_ref[...], preferred_element_type=jnp.float32)
    h_ref[...] = h
    a2_ref[...] = jnp.dot(h, a8_ref[...], preferred_element_type=jnp.float32)


def _prep(x, W, A8):
    return pl.pallas_call(
        _prep_body,
        grid=(N // BR,),
        in_specs=[pl.BlockSpec((BR, D), lambda i: (i, 0)),
                  pl.BlockSpec((D, D), lambda i: (0, 0)),
                  pl.BlockSpec((D, 8), lambda i: (0, 0))],
        out_specs=[pl.BlockSpec((BR, D), lambda i: (i, 0)),
                   pl.BlockSpec((BR, 8), lambda i: (i, 0))],
        out_shape=[jax.ShapeDtypeStruct((N, D), jnp.float32),
                   jax.ShapeDtypeStruct((N, 8), jnp.float32)],
    )(x, W, A8)


def _fin_body(op_ref, dn_ref, h_ref, a2_ref, p_ref, o_ref):
    s = a2_ref[:, 0:1] + a2_ref[:, 1:2]
    ws = jnp.exp(jnp.maximum(s, 0.2 * s))
    num = op_ref[0] + op_ref[1] + ws * h_ref[...]
    den = jnp.sum(dn_ref[...], axis=1, keepdims=True) + ws + 1e-16
    y = (num / den + p_ref[0:1, :]) * (p_ref[1:2, :] * BN_SCALE) + p_ref[2:3, :]
    o_ref[...] = jnp.maximum(y, 0.0)


def _finalize(outp, dent, h, a2, bgbe):
    return pl.pallas_call(
        _fin_body,
        grid=(N // BR,),
        in_specs=[pl.BlockSpec((2, BR, D), lambda i: (0, i, 0)),
                  pl.BlockSpec((BR, NW), lambda i: (i, 0)),
                  pl.BlockSpec((BR, D), lambda i: (i, 0)),
                  pl.BlockSpec((BR, 8), lambda i: (i, 0)),
                  pl.BlockSpec((3, D), lambda i: (0, 0))],
        out_specs=pl.BlockSpec((BR, D), lambda i: (i, 0)),
        out_shape=jax.ShapeDtypeStruct((N, D), jnp.float32),
    )(outp, dent, h, a2, bgbe)


def kernel(last_x, edge_index, W0, a_src0, a_dst0, b0, g0, be0,
           W1, a_src1, a_dst1, b1, g1, be1):
    ei = edge_index.astype(jnp.int32)
    pad = jnp.full((E_PAD - E,), N, jnp.int32)
    cap_h = NS * CPT_H * K

    def lay(row):
        e = jnp.concatenate([row, pad])
        eh = e[:cap_h].reshape(NS, CPT_H, K)
        el = e[cap_h:].reshape(NS, CPT_L, K)
        h0 = HEAVY * NS
        l0 = (1 - HEAVY) * NS
        out = jnp.full((NW, CPT, K), N, jnp.int32)
        return out.at[h0:h0 + NS, :CPT_H].set(eh).at[l0:l0 + NS, :CPT_L].set(el)

    srcs = lay(ei[0])
    dsts = lay(ei[1])
    z = jnp.zeros((NPAD, D), jnp.float32)
    x = last_x
    for (W, a_s, a_d, b, g, be) in ((W0, a_src0, a_dst0, b0, g0, be0),
                                    (W1, a_src1, a_dst1, b1, g1, be1)):
        A8 = jnp.zeros((D, 8), jnp.float32).at[:, 0].set(a_s).at[:, 1].set(a_d)
        h, a2 = _prep(x, W, A8)
        h_pad = jnp.pad(h, ((0, NPAD - N), (0, 0)))
        a2p = jnp.pad(a2, ((0, NPAD - N), (0, 0)))
        wv, denh = _sc_w(a2p[:, 0], a2p[:, 1], srcs, dsts)
        outp = _sc_scatter(h_pad, srcs, dsts, wv, z)
        dent = denh.T
        bgbe = jnp.stack([b, g, be])
        x = _finalize(outp, dent, h, a2, bgbe)
    return x


# speedup vs baseline: 1.0127x; 1.0127x over previous
import math

import jax
import jax.numpy as jnp
from jax import lax
from jax.experimental import pallas as pl
from jax.experimental.pallas import tpu as pltpu
from jax.experimental.pallas import tpu_sc as plsc

N = 10000
D = 128
E = 320000
EPS_BN = 1e-12
BN_SCALE = 1.0 / math.sqrt(1.0 + EPS_BN)

NC, NS, L = 2, 16, 16
NW = NC * NS
K = 112
CPT_H = 120
CPT_L = 60
CPT = CPT_H
HEAVY = 1
E_PAD = NS * (CPT_H + CPT_L) * K
RPT = 640
NPAD = NS * RPT
NBUF = 3
SR = 6
BR = 400


def _sc_w_body(asrc_hbm, adst_hbm, src_hbm, dst_hbm,
               w_hbm, denh_hbm,
               asrc_v, adst_v, src_v, dst_v, w_v, den_l):
    c = lax.axis_index("c")
    s = lax.axis_index("s")
    wid = c * NS + s

    pltpu.sync_copy(asrc_hbm, asrc_v)
    pltpu.sync_copy(adst_hbm, adst_v)
    pltpu.sync_copy(src_hbm.at[wid], src_v)
    pltpu.sync_copy(dst_hbm.at[wid], dst_v)

    zero16 = jnp.zeros((L,), jnp.float32)

    def zbody(i, carry):
        den_l[pl.ds(i * L, L)] = zero16
        return carry
    lax.fori_loop(0, NPAD // L, zbody, 0)

    cpt = jnp.where(c == HEAVY, CPT_H, CPT_L)

    def wbody(j, carry):
        @pl.when(j < cpt)
        def _():
            for v in range(K // L):
                sl = pl.ds(v * L, L)
                si = src_v[j, sl]
                di = dst_v[j, sl]
                a = (plsc.load_gather(asrc_v, [si])
                     + plsc.load_gather(adst_v, [di]))
                w = jnp.exp(jnp.maximum(a, 0.2 * a))
                w_v[j, sl] = w
                plsc.addupdate_scatter(den_l, [di], w)
        return carry
    lax.fori_loop(0, CPT, wbody, 0)

    pltpu.sync_copy(w_v, w_hbm.at[wid])
    pltpu.sync_copy(den_l, denh_hbm.at[wid])


_sc_w = pl.kernel(
    _sc_w_body,
    out_type=(jax.ShapeDtypeStruct((NW, CPT, K), jnp.float32),
              jax.ShapeDtypeStruct((NW, NPAD), jnp.float32)),
    mesh=plsc.VectorSubcoreMesh(core_axis_name="c", subcore_axis_name="s",
                                num_cores=NC, num_subcores=NS),
    compiler_params=pltpu.CompilerParams(needs_layout_passes=False),
    scratch_types=[
        pltpu.VMEM((NPAD,), jnp.float32),
        pltpu.VMEM((NPAD,), jnp.float32),
        pltpu.VMEM((CPT, K), jnp.int32),
        pltpu.VMEM((CPT, K), jnp.int32),
        pltpu.VMEM((CPT, K), jnp.float32),
        pltpu.VMEM((NPAD,), jnp.float32),
    ],
)


def _sc_scatter_body(h_hbm, src_hbm, dst_hbm, w_hbm, z_hbm,
                     outp_hbm,
                     src_r, dst_r, w_r, hbuf, out_acc, *sems):
    gsem = sems[0:NBUF]
    ssem = sems[NBUF:2 * NBUF]
    stsem = sems[2 * NBUF:2 * NBUF + SR]
    c = lax.axis_index("c")
    s = lax.axis_index("s")
    wid = c * NS + s
    r0 = s * RPT
    cpt = jnp.where(c == HEAVY, CPT_H, CPT_L)

    pltpu.sync_copy(z_hbm.at[pl.ds(r0, RPT)], out_acc.at[pl.ds(r0, RPT)])
    plsc.subcore_barrier()

    def stage(j, sr):
        pltpu.async_copy(src_hbm.at[wid, j], src_r.at[sr], stsem[sr])
        pltpu.async_copy(dst_hbm.at[wid, j], dst_r.at[sr], stsem[sr])
        pltpu.async_copy(w_hbm.at[wid, j], w_r.at[sr], stsem[sr])

    def wait_stage(j, sr):
        pltpu.make_async_copy(src_hbm.at[wid, j], src_r.at[sr], stsem[sr]).wait()
        pltpu.make_async_copy(dst_hbm.at[wid, j], dst_r.at[sr], stsem[sr]).wait()
        pltpu.make_async_copy(w_hbm.at[wid, j], w_r.at[sr], stsem[sr]).wait()

    def chunk(j, b, sr):
        bn = (b + 2) % NBUF

        @pl.when(j < cpt)
        def _():
            pltpu.make_async_copy(h_hbm.at[src_r.at[sr]], hbuf.at[b],
                                  gsem[b]).wait()

            def sbody(g, carry):
                wv = w_r[sr, pl.ds(g * L, L)]
                for t in range(L):
                    wk = wv[t]
                    row = g * L + t
                    for u in range(D // L):
                        sl = pl.ds(u * L, L)
                        hbuf[b, row, sl] = hbuf[b, row, sl] * wk
                return carry
            lax.fori_loop(0, K // L, sbody, 0)

        @pl.when((j >= 1) & (j < cpt))
        def _():
            pltpu.make_async_copy(hbuf.at[bn],
                                  out_acc.at[dst_r.at[(sr + SR - 1) % SR]],
                                  ssem[bn]).wait()

        @pl.when(j + 3 < cpt)
        def _():
            stage(j + 3, (sr + 3) % SR)

        @pl.when(j + 2 < cpt)
        def _():
            wait_stage(j + 2, (sr + 2) % SR)
            pltpu.async_copy(h_hbm.at[src_r.at[(sr + 2) % SR]], hbuf.at[bn],
                             gsem[bn])

        @pl.when(j < cpt)
        def _():
            pltpu.async_copy(hbuf.at[b], out_acc.at[dst_r.at[sr]], ssem[b],
                             add=True)

    stage(0, 0)
    stage(1, 1)
    stage(2, 2)
    wait_stage(0, 0)
    pltpu.async_copy(h_hbm.at[src_r.at[0]], hbuf.at[0], gsem[0])
    wait_stage(1, 1)
    pltpu.async_copy(h_hbm.at[src_r.at[1]], hbuf.at[1], gsem[1])

    def cbody(t, carry):
        j = SR * t
        for k in range(SR):
            chunk(j + k, k % NBUF, k)
        return carry
    lax.fori_loop(0, CPT // SR, cbody, 0)

    assert (CPT_H - 1) % NBUF == (CPT_L - 1) % NBUF
    assert (CPT_H - 1) % SR == (CPT_L - 1) % SR
    pltpu.make_async_copy(hbuf.at[(CPT_H - 1) % NBUF],
                          out_acc.at[dst_r.at[(CPT_H - 1) % SR]],
                          ssem[(CPT_H - 1) % NBUF]).wait()
    plsc.subcore_barrier()

    pltpu.sync_copy(out_acc.at[pl.ds(r0, RPT)], outp_hbm.at[c, pl.ds(r0, RPT)])


_sc_scatter = pl.kernel(
    _sc_scatter_body,
    out_type=jax.ShapeDtypeStruct((NC, NPAD, D), jnp.float32),
    mesh=plsc.VectorSubcoreMesh(core_axis_name="c", subcore_axis_name="s",
                                num_cores=NC, num_subcores=NS),
    compiler_params=pltpu.CompilerParams(needs_layout_passes=False),
    scratch_types=[
        pltpu.VMEM((SR, K), jnp.int32),
        pltpu.VMEM((SR, K), jnp.int32),
        pltpu.VMEM((SR, K), jnp.float32),
        pltpu.VMEM((NBUF, K, D), jnp.float32),
        pltpu.VMEM_SHARED((NPAD, D), jnp.float32),
    ] + [pltpu.SemaphoreType.DMA] * (2 * NBUF + SR),
)


def _prep_body(x_ref, w_ref, a8_ref, h_ref, a2_ref):
    h = jnp.dot(x_ref[...], w_ref[...], preferred_element_type=jnp.float32)
    h_ref[...] = h
    a2_ref[...] = jnp.dot(h, a8_ref[...], preferred_element_type=jnp.float32)


def _prep(x, W, A8):
    return pl.pallas_call(
        _prep_body,
        grid=(N // BR,),
        in_specs=[pl.BlockSpec((BR, D), lambda i: (i, 0)),
                  pl.BlockSpec((D, D), lambda i: (0, 0)),
                  pl.BlockSpec((D, 8), lambda i: (0, 0))],
        out_specs=[pl.BlockSpec((BR, D), lambda i: (i, 0)),
                   pl.BlockSpec((BR, 8), lambda i: (i, 0))],
        out_shape=[jax.ShapeDtypeStruct((N, D), jnp.float32),
                   jax.ShapeDtypeStruct((N, 8), jnp.float32)],
    )(x, W, A8)


def _fin_body(op_ref, dn_ref, h_ref, a2_ref, p_ref, o_ref):
    s = a2_ref[:, 0:1] + a2_ref[:, 1:2]
    ws = jnp.exp(jnp.maximum(s, 0.2 * s))
    num = op_ref[0] + op_ref[1] + ws * h_ref[...]
    den = jnp.sum(dn_ref[...], axis=1, keepdims=True) + ws + 1e-16
    y = (num / den + p_ref[0:1, :]) * (p_ref[1:2, :] * BN_SCALE) + p_ref[2:3, :]
    o_ref[...] = jnp.maximum(y, 0.0)


def _finalize(outp, dent, h, a2, bgbe):
    return pl.pallas_call(
        _fin_body,
        grid=(N // BR,),
        in_specs=[pl.BlockSpec((2, BR, D), lambda i: (0, i, 0)),
                  pl.BlockSpec((BR, NW), lambda i: (i, 0)),
                  pl.BlockSpec((BR, D), lambda i: (i, 0)),
                  pl.BlockSpec((BR, 8), lambda i: (i, 0)),
                  pl.BlockSpec((3, D), lambda i: (0, 0))],
        out_specs=pl.BlockSpec((BR, D), lambda i: (i, 0)),
        out_shape=jax.ShapeDtypeStruct((N, D), jnp.float32),
    )(outp, dent, h, a2, bgbe)


def kernel(last_x, edge_index, W0, a_src0, a_dst0, b0, g0, be0,
           W1, a_src1, a_dst1, b1, g1, be1):
    ei = edge_index.astype(jnp.int32)
    pad = jnp.full((E_PAD - E,), N, jnp.int32)
    cap_h = NS * CPT_H * K

    def lay(row):
        e = jnp.concatenate([row, pad])
        eh = e[:cap_h].reshape(NS, CPT_H, K)
        el = e[cap_h:].reshape(NS, CPT_L, K)
        h0 = HEAVY * NS
        l0 = (1 - HEAVY) * NS
        out = jnp.full((NW, CPT, K), N, jnp.int32)
        return out.at[h0:h0 + NS, :CPT_H].set(eh).at[l0:l0 + NS, :CPT_L].set(el)

    srcs = lay(ei[0])
    dsts = lay(ei[1])
    z = jnp.zeros((NPAD, D), jnp.float32)
    x = last_x
    for (W, a_s, a_d, b, g, be) in ((W0, a_src0, a_dst0, b0, g0, be0),
                                    (W1, a_src1, a_dst1, b1, g1, be1)):
        A8 = jnp.zeros((D, 8), jnp.float32).at[:, 0].set(a_s).at[:, 1].set(a_d)
        h, a2 = _prep(x, W, A8)
        h_pad = jnp.pad(h, ((0, NPAD - N), (0, 0)))
        a2p = jnp.pad(a2, ((0, NPAD - N), (0, 0)))
        wv, denh = _sc_w(a2p[:, 0], a2p[:, 1], srcs, dsts)
        outp = _sc_scatter(h_pad, srcs, dsts, wv, z)
        dent = denh.T
        bgbe = jnp.stack([b, g, be])
        x = _finalize(outp, dent, h, a2, bgbe)
    return x
